# baseline (speedup 1.0000x reference)
import functools

import jax
import jax.numpy as jnp
from jax import lax
from jax.experimental import pallas as pl
from jax.experimental.pallas import tpu as pltpu
from jax.experimental.pallas import tpu_sc as plsc

NUM_FEATURES = 26
VOCAB = 100000
EMBED_DIM = 32
BATCH = 16384
HEADS = 8
HEAD_DIM = 64
D_ALL = NUM_FEATURES * EMBED_DIM

NC, NS = 2, 16
NW = NC * NS
ROWS = NUM_FEATURES * BATCH
RPW = ROWS // NW
CH = 256
NCH = RPW // CH
T128_ROWS = NUM_FEATURES * VOCAB * EMBED_DIM // 128
O128_ROWS = ROWS * EMBED_DIM // 128
ORPW = O128_ROWS // NW
OCH = CH * EMBED_DIM // 128

BPW = BATCH // NW

_mesh = plsc.VectorSubcoreMesh(core_axis_name="c", subcore_axis_name="s")


@functools.partial(
    pl.kernel,
    mesh=_mesh,
    compiler_params=pltpu.CompilerParams(
        use_tc_tiling_on_sc=True, needs_layout_passes=False),
    out_type=jax.ShapeDtypeStruct((O128_ROWS, 128), jnp.float32),
    scratch_types=[
        pltpu.VMEM((RPW,), jnp.int32),
        pltpu.VMEM((RPW,), jnp.int32),
        pltpu.VMEM((RPW,), jnp.int32),
        pltpu.VMEM((CH, 128), jnp.float32),
        pltpu.VMEM((CH, 128), jnp.float32),
        pltpu.VMEM((OCH, 128), jnp.float32),
        pltpu.VMEM((OCH, 128), jnp.float32),
        pltpu.SemaphoreType.DMA,
        pltpu.SemaphoreType.DMA,
        pltpu.SemaphoreType.DMA,
    ],
)
def _sc_gather(tab_hbm, nsf_hbm, out_hbm, raw_v, g4_v, off_v,
               buf0, buf1, ob0, ob1, isem, gsem, wsem):
    wid = lax.axis_index("s") * NC + lax.axis_index("c")
    b0 = wid * BPW
    obase = wid * ORPW
    loads = []
    for f in range(NUM_FEATURES):
        loads.append(pltpu.async_copy(
            nsf_hbm.at[pl.ds(f * BATCH + b0, BPW)],
            raw_v.at[pl.ds(f * BPW, BPW)], isem))
    for ld in loads:
        ld.wait()

    iota = lax.iota(jnp.int32, 16)
    iota26 = iota * NUM_FEATURES

    def build(k, carry):
        for f in range(NUM_FEATURES):
            v = raw_v[pl.ds(f * BPW + k * 16, 16)]
            pos = iota26 + (k * (16 * NUM_FEATURES) + f)
            plsc.store_scatter(g4_v, [pos],
                               (v >> 2) + (f * (VOCAB // 4)))
            plsc.store_scatter(off_v, [pos], (v & 3) << 5)
        return carry
    lax.fori_loop(0, BPW // 16, build, 0)

    bufs = [buf0, buf1]
    obufs = [ob0, ob1]

    def chunk(s, carry):
        for j in range(2):
            c = s * 2 + j
            buf, obuf = bufs[j], obufs[j]
            pltpu.async_copy(
                tab_hbm.at[g4_v.at[pl.ds(c * CH, CH)]], buf, gsem).wait()

            def extract(g, cc):
                rvec = iota + g * 16
                offv = plsc.load_gather(off_v, [c * CH + rvec])
                wrow0 = g * 4
                for d in range(EMBED_DIM):
                    w = iota * EMBED_DIM + d
                    vals = plsc.load_gather(buf, [rvec, offv + d])
                    plsc.store_scatter(
                        obuf, [(w >> 7) + wrow0, w & 127], vals)
                return cc
            lax.fori_loop(0, CH // 16, extract, 0)
            pltpu.async_copy(
                obuf, out_hbm.at[pl.ds(obase + c * OCH, OCH)], wsem).wait()
        return carry
    lax.fori_loop(0, NCH // 2, chunk, 0)


def _mm_body(x_ref, w_ref, o_ref):
    o_ref[...] = jax.lax.dot_general(
        w_ref[...], x_ref[...],
        dimension_numbers=(((1,), (1,)), ((), ())),
        preferred_element_type=jnp.float32)


BB = 1024

_mm = pl.pallas_call(
    _mm_body,
    grid=(BATCH // BB,),
    in_specs=[
        pl.BlockSpec((BB, D_ALL), lambda i: (i, 0)),
        pl.BlockSpec((HEADS * HEAD_DIM, D_ALL), lambda i: (0, 0)),
    ],
    out_specs=pl.BlockSpec((HEADS * HEAD_DIM, BB), lambda i: (0, i)),
    out_shape=jax.ShapeDtypeStruct((HEADS * HEAD_DIM, BATCH), jnp.float32),
)


def kernel(non_seq_features, tables, W_user, W_item):
    t128 = tables.reshape(T128_ROWS, 128)
    nsf = non_seq_features.reshape(ROWS)
    gathered = _sc_gather(t128, nsf)
    w_all = jnp.concatenate([W_user, W_item], axis=0)
    wbd_t = jax.scipy.linalg.block_diag(*[w_all[j] for j in range(HEADS)])
    out_t = _mm(gathered.reshape(BATCH, D_ALL), wbd_t)
    return out_t.reshape(HEADS, HEAD_DIM, BATCH).transpose(2, 0, 1)


# speedup vs baseline: 1.4410x; 1.4410x over previous
import functools

import jax
import jax.numpy as jnp
from jax import lax
from jax.experimental import pallas as pl
from jax.experimental.pallas import tpu as pltpu
from jax.experimental.pallas import tpu_sc as plsc

NUM_FEATURES = 26
VOCAB = 100000
EMBED_DIM = 32
BATCH = 16384
HEADS = 8
SPLIT = 104
HEAD_DIM = 64
D_ALL = NUM_FEATURES * EMBED_DIM

NC, NS = 2, 16
NW = NC * NS
ROWS = NUM_FEATURES * BATCH
RPW = ROWS // NW
CH = 1024
NCH = RPW // CH

_mesh = plsc.VectorSubcoreMesh(core_axis_name="c", subcore_axis_name="s")


BPW = BATCH // NW


@functools.partial(
    pl.kernel,
    mesh=_mesh,
    compiler_params=pltpu.CompilerParams(
        use_tc_tiling_on_sc=False, needs_layout_passes=False),
    out_type=jax.ShapeDtypeStruct((ROWS, EMBED_DIM), jnp.float32),
    scratch_types=[
        pltpu.VMEM((RPW,), jnp.int32),
        pltpu.VMEM((RPW,), jnp.int32),
        pltpu.VMEM((CH, EMBED_DIM), jnp.float32),
        pltpu.VMEM((CH, EMBED_DIM), jnp.float32),
        pltpu.SemaphoreType.DMA,
        pltpu.SemaphoreType.DMA,
        pltpu.SemaphoreType.DMA,
    ],
)
def _sc_gather(tab_hbm, nsf_hbm, out_hbm, raw_v, gidx_v, buf0, buf1,
               isem, gsem, wsem):
    wid = lax.axis_index("s") * NC + lax.axis_index("c")
    b0 = wid * BPW
    base = wid * RPW
    loads = []
    for f in range(NUM_FEATURES):
        loads.append(pltpu.async_copy(
            nsf_hbm.at[pl.ds(f * BATCH + b0, BPW)],
            raw_v.at[pl.ds(f * BPW, BPW)], isem))
    for ld in loads:
        ld.wait()

    iota26 = lax.iota(jnp.int32, 16) * NUM_FEATURES

    def build(k, carry):
        for f in range(NUM_FEATURES):
            v = raw_v[pl.ds(f * BPW + k * 16, 16)] + (f * VOCAB)
            pos = iota26 + (k * (16 * NUM_FEATURES) + f)
            plsc.store_scatter(gidx_v, [pos], v)
        return carry
    lax.fori_loop(0, BPW // 16, build, 0)

    writes = [None, None]
    bufs = [buf0, buf1]
    for c in range(NCH):
        bb = c % 2
        if writes[bb] is not None:
            writes[bb].wait()
        pltpu.async_copy(
            tab_hbm.at[gidx_v.at[pl.ds(c * CH, CH)]], bufs[bb], gsem).wait()
        writes[bb] = pltpu.async_copy(
            bufs[bb], out_hbm.at[pl.ds(base + c * CH, CH)], wsem)
    for w in writes:
        if w is not None:
            w.wait()


def _mm_body(x_ref, w_ref, o_ref):
    o_ref[...] = jax.lax.dot_general(
        w_ref[...], x_ref[...],
        dimension_numbers=(((1,), (1,)), ((), ())),
        preferred_element_type=jnp.float32)


BB = 1024

_mm = pl.pallas_call(
    _mm_body,
    grid=(BATCH // BB,),
    in_specs=[
        pl.BlockSpec((BB, D_ALL), lambda i: (i, 0)),
        pl.BlockSpec((HEADS * HEAD_DIM, D_ALL), lambda i: (0, 0)),
    ],
    out_specs=pl.BlockSpec((HEADS * HEAD_DIM, BB), lambda i: (0, i)),
    out_shape=jax.ShapeDtypeStruct((HEADS * HEAD_DIM, BATCH), jnp.float32),
)


def kernel(non_seq_features, tables, W_user, W_item):
    flat = tables.reshape(NUM_FEATURES * VOCAB, EMBED_DIM)
    nsf = non_seq_features.reshape(ROWS)
    gathered = _sc_gather(flat, nsf)
    w_all = jnp.concatenate([W_user, W_item], axis=0)
    wbd_t = jax.scipy.linalg.block_diag(*[w_all[j] for j in range(HEADS)])
    out_t = _mm(gathered.reshape(BATCH, D_ALL), wbd_t)
    return out_t.reshape(HEADS, HEAD_DIM, BATCH).transpose(2, 0, 1)
